# baseline (speedup 1.0000x reference)
import numpy as np

import jax
import jax.numpy as jnp
from jax.experimental import pallas as pl
from jax.experimental.pallas import tpu as pltpu

_B, _C, _H, _W = 4, 3, 224, 224

_U32 = np.uint32


def _rotl(x, d):
    d = _U32(d)
    return ((x << d) | (x >> _U32(32 - d))).astype(np.uint32)


def _threefry2x32(k1, k2, x1, x2):
    rotations = ((13, 15, 26, 6), (17, 29, 16, 24))
    ks = (_U32(k1), _U32(k2), _U32(k1) ^ _U32(k2) ^ _U32(0x1BD11BDA))
    x = [np.asarray(x1, np.uint32).copy(), np.asarray(x2, np.uint32).copy()]
    x[0] = (x[0] + ks[0]).astype(np.uint32)
    x[1] = (x[1] + ks[1]).astype(np.uint32)

    def rounds(x, rots):
        for r in rots:
            x[0] = (x[0] + x[1]).astype(np.uint32)
            x[1] = x[0] ^ _rotl(x[1], r)
        return x

    for i, (ka, kb) in enumerate(((1, 2), (2, 0), (0, 1), (1, 2), (2, 0))):
        x = rounds(x, rotations[i % 2])
        x[0] = (x[0] + ks[ka]).astype(np.uint32)
        x[1] = (x[1] + ks[kb] + _U32(i + 1)).astype(np.uint32)
    return x[0], x[1]


def _split(key, num):
    b1, b2 = _threefry2x32(key[0], key[1],
                           np.zeros(num, np.uint32),
                           np.arange(num, dtype=np.uint32))
    return [(b1[i], b2[i]) for i in range(num)]


def _fold_in(key, data):
    b1, b2 = _threefry2x32(key[0], key[1],
                           np.zeros(1, np.uint32),
                           np.full(1, data, np.uint32))
    return (b1[0], b2[0])


def _random_bits32(key, n):
    b1, b2 = _threefry2x32(key[0], key[1],
                           np.zeros(n, np.uint32),
                           np.arange(n, dtype=np.uint32))
    return b1 ^ b2


def _randint(key, n, minval, maxval):
    k1, k2 = _split(key, 2)
    higher_bits = _random_bits32(k1, n)
    lower_bits = _random_bits32(k2, n)
    span = _U32(maxval - minval)
    multiplier = _U32((2 ** 16) % int(span))
    multiplier = _U32((int(multiplier) * int(multiplier)) % int(span))
    offset = ((higher_bits % span) * multiplier + (lower_bits % span)) % span
    return (np.int32(minval) + offset.astype(np.int32)).astype(np.int32)


def _snow_mask_np(seed, B, H, W):
    n = int(H * W * 0.015)
    key = (_U32(seed >> 32), _U32(seed & 0xFFFFFFFF))
    Y = np.arange(H)
    X = np.arange(W)
    masks = []
    for b in range(B):
        kb = _fold_in(key, b)
        k1, k2, k3 = _split(kb, 3)
        ys = _randint(k1, n, 0, H)
        xs = _randint(k2, n, 0, W)
        rs = _randint(k3, n, 1, 4)
        yy = np.abs(Y[None, :] - ys[:, None]) <= rs[:, None]
        xx = np.abs(X[None, :] - xs[:, None]) <= rs[:, None]
        masks.append(np.any(yy[:, :, None] & xx[:, None, :], axis=0))
    return np.stack(masks)


_MASK_I8 = _snow_mask_np(42, _B, _H, _W).astype(np.int8)[:, None, :, :]


def _gauss_weights() -> np.ndarray:
    k, sigma = 5, 1.5
    coords = np.arange(k, dtype=np.float32) - k // 2
    g = np.exp(-coords.astype(np.float32) ** 2 / np.float32(2.0 * sigma**2))
    return (g / g.sum()).astype(np.float32)


_G = _gauss_weights()


def _band_matrix() -> np.ndarray:
    g = _gauss_weights()
    a = np.zeros((_H, _H), np.float32)
    for d in range(-2, 3):
        idx = np.arange(max(0, -d), _H - max(0, d))
        a[idx, idx + d] = g[d + 2]
    return a


_A = _band_matrix()


def _snow_blur_kernel(x_hbm, m_ref, a_ref, o_hbm, xv, ov, in_sems, out_sems):
    for i in range(_B):
        pltpu.make_async_copy(x_hbm.at[i], xv.at[i], in_sems.at[i]).start()
    a = a_ref[...]
    for i in range(_B):
        pltpu.make_async_copy(x_hbm.at[i], xv.at[i], in_sems.at[i]).wait()
        nz = m_ref[i, 0] != 0
        for c in range(_C):
            s = jnp.where(nz, jnp.float32(0.95), xv[i, c]).astype(jnp.bfloat16)
            y = jax.lax.dot_general(a, s, (((1,), (0,)), ((), ())),
                                    preferred_element_type=jnp.float32)
            z = jax.lax.dot_general(y.astype(jnp.bfloat16), a,
                                    (((1,), (0,)), ((), ())),
                                    preferred_element_type=jnp.float32)
            ov[i, c] = jnp.clip(z, 0.0, 1.0)
        pltpu.make_async_copy(ov.at[i], o_hbm.at[i], out_sems.at[i]).start()
    for i in range(_B):
        pltpu.make_async_copy(ov.at[i], o_hbm.at[i], out_sems.at[i]).wait()


def kernel(x):
    mask = jnp.asarray(_MASK_I8)
    band = jnp.asarray(_A.astype(np.float32)).astype(jnp.bfloat16)
    return pl.pallas_call(
        _snow_blur_kernel,
        in_specs=[
            pl.BlockSpec(memory_space=pl.ANY),
            pl.BlockSpec(memory_space=pltpu.MemorySpace.VMEM),
            pl.BlockSpec(memory_space=pltpu.MemorySpace.VMEM),
        ],
        out_specs=pl.BlockSpec(memory_space=pl.ANY),
        out_shape=jax.ShapeDtypeStruct((_B, _C, _H, _W), jnp.float32),
        scratch_shapes=[
            pltpu.MemorySpace.VMEM((_B, _C, _H, _W), jnp.float32),
            pltpu.MemorySpace.VMEM((_B, _C, _H, _W), jnp.float32),
            pltpu.SemaphoreType.DMA((_B,)),
            pltpu.SemaphoreType.DMA((_B,)),
        ],
    )(x, mask, band)


# speedup vs baseline: 1.3039x; 1.3039x over previous
import numpy as np

import jax
import jax.numpy as jnp
from jax.experimental import pallas as pl

_B, _C, _H, _W = 4, 3, 224, 224

_U32 = np.uint32


def _rotl(x, d):
    d = _U32(d)
    return ((x << d) | (x >> _U32(32 - d))).astype(np.uint32)


def _threefry2x32(k1, k2, x1, x2):
    rotations = ((13, 15, 26, 6), (17, 29, 16, 24))
    ks = (_U32(k1), _U32(k2), _U32(k1) ^ _U32(k2) ^ _U32(0x1BD11BDA))
    x = [np.asarray(x1, np.uint32).copy(), np.asarray(x2, np.uint32).copy()]
    x[0] = (x[0] + ks[0]).astype(np.uint32)
    x[1] = (x[1] + ks[1]).astype(np.uint32)

    def rounds(x, rots):
        for r in rots:
            x[0] = (x[0] + x[1]).astype(np.uint32)
            x[1] = x[0] ^ _rotl(x[1], r)
        return x

    for i, (ka, kb) in enumerate(((1, 2), (2, 0), (0, 1), (1, 2), (2, 0))):
        x = rounds(x, rotations[i % 2])
        x[0] = (x[0] + ks[ka]).astype(np.uint32)
        x[1] = (x[1] + ks[kb] + _U32(i + 1)).astype(np.uint32)
    return x[0], x[1]


def _split(key, num):
    b1, b2 = _threefry2x32(key[0], key[1],
                           np.zeros(num, np.uint32),
                           np.arange(num, dtype=np.uint32))
    return [(b1[i], b2[i]) for i in range(num)]


def _fold_in(key, data):
    b1, b2 = _threefry2x32(key[0], key[1],
                           np.zeros(1, np.uint32),
                           np.full(1, data, np.uint32))
    return (b1[0], b2[0])


def _random_bits32(key, n):
    b1, b2 = _threefry2x32(key[0], key[1],
                           np.zeros(n, np.uint32),
                           np.arange(n, dtype=np.uint32))
    return b1 ^ b2


def _randint(key, n, minval, maxval):
    k1, k2 = _split(key, 2)
    higher_bits = _random_bits32(k1, n)
    lower_bits = _random_bits32(k2, n)
    span = _U32(maxval - minval)
    multiplier = _U32((2 ** 16) % int(span))
    multiplier = _U32((int(multiplier) * int(multiplier)) % int(span))
    offset = ((higher_bits % span) * multiplier + (lower_bits % span)) % span
    return (np.int32(minval) + offset.astype(np.int32)).astype(np.int32)


def _snow_mask_np(seed, B, H, W):
    n = int(H * W * 0.015)
    key = (_U32(seed >> 32), _U32(seed & 0xFFFFFFFF))
    Y = np.arange(H)
    X = np.arange(W)
    masks = []
    for b in range(B):
        kb = _fold_in(key, b)
        k1, k2, k3 = _split(kb, 3)
        ys = _randint(k1, n, 0, H)
        xs = _randint(k2, n, 0, W)
        rs = _randint(k3, n, 1, 4)
        yy = np.abs(Y[None, :] - ys[:, None]) <= rs[:, None]
        xx = np.abs(X[None, :] - xs[:, None]) <= rs[:, None]
        masks.append(np.any(yy[:, :, None] & xx[:, None, :], axis=0))
    return np.stack(masks)


_MASK_I8 = _snow_mask_np(42, _B, _H, _W).astype(np.int8)[:, None, :, :]


def _gauss_weights() -> np.ndarray:
    k, sigma = 5, 1.5
    coords = np.arange(k, dtype=np.float32) - k // 2
    g = np.exp(-coords.astype(np.float32) ** 2 / np.float32(2.0 * sigma**2))
    return (g / g.sum()).astype(np.float32)


_G = _gauss_weights()


def _band_matrix() -> np.ndarray:
    g = _gauss_weights()
    a = np.zeros((_H, _H), np.float32)
    for d in range(-2, 3):
        idx = np.arange(max(0, -d), _H - max(0, d))
        a[idx, idx + d] = g[d + 2]
    return a


_A = _band_matrix()


def _snow_blur_kernel(x_ref, m_ref, a_ref, o_ref):
    a = a_ref[...]
    for b in range(2):
      nz = m_ref[b, 0] != 0
      for c in range(_C):
        s = jnp.where(nz, jnp.bfloat16(0.95), x_ref[b, c].astype(jnp.bfloat16))
        y = jax.lax.dot_general(a, s, (((1,), (0,)), ((), ())),
                                preferred_element_type=jnp.float32)
        z = jax.lax.dot_general(y.astype(jnp.bfloat16), a,
                                (((1,), (0,)), ((), ())),
                                preferred_element_type=jnp.float32)
        o_ref[b, c] = jnp.clip(z, 0.0, 1.0)


def kernel(x):
    mask = jnp.asarray(_MASK_I8)
    band = jnp.asarray(_A.astype(np.float32)).astype(jnp.bfloat16)
    return pl.pallas_call(
        _snow_blur_kernel,
        grid=(_B // 2,),
        in_specs=[
            pl.BlockSpec((2, _C, _H, _W), lambda b: (b, 0, 0, 0)),
            pl.BlockSpec((2, 1, _H, _W), lambda b: (b, 0, 0, 0)),
            pl.BlockSpec((_H, _H), lambda b: (0, 0)),
        ],
        out_specs=pl.BlockSpec((2, _C, _H, _W), lambda b: (b, 0, 0, 0)),
        out_shape=jax.ShapeDtypeStruct((_B, _C, _H, _W), jnp.float32),
    )(x, mask, band)
